# baseline (speedup 1.0000x reference)
import functools

import jax
import jax.numpy as jnp
from jax import lax
from jax.experimental import pallas as pl
from jax.experimental.pallas import tpu as pltpu
from jax.experimental.pallas import tpu_sc as plsc

N = 10000
E = 320000
R = 8
F = 128
RN = R * N
HR = RN // F

NC = 2
NS = 16
NW = NC * NS

EPT_A = E // NS
CH_A = 2000
NCH_A = EPT_A // CH_A

EPT = E // NW
CH_W = 2000
NCH_W = EPT // CH_W
CH_C = 125
NCH_C = EPT // CH_C

BN = 400
NB = N // BN

_mesh = plsc.VectorSubcoreMesh(
    core_axis_name="c", subcore_axis_name="s", num_cores=NC, num_subcores=NS)
_sc_params = pltpu.CompilerParams(needs_layout_passes=False)

f32 = jnp.float32
i32 = jnp.int32


@functools.partial(
    pl.kernel,
    out_type=(
        jax.ShapeDtypeStruct((NS, RN), f32),
        jax.ShapeDtypeStruct((NS, RN), f32),
        jax.ShapeDtypeStruct((E,), i32),
        jax.ShapeDtypeStruct((E,), i32),
    ),
    mesh=_mesh,
    compiler_params=_sc_params,
    scratch_types=(
        pltpu.VMEM((RN,), f32),
        pltpu.VMEM((EPT_A,), i32),
        pltpu.VMEM((EPT_A,), i32),
        pltpu.VMEM((CH_A,), i32),
    ),
)
def _hist_kernel(src_hbm, dst_hbm, et_hbm, zeros_hbm,
                 histo_hbm, histi_hbm, gsrc_hbm, gdst_hbm,
                 hist_v, node_all, et_all, g_c):
    c = lax.axis_index("c")
    s = lax.axis_index("s")
    ones = jnp.ones((16,), f32)

    def tile_work(node_hbm, g_hbm, hist_out_hbm):
        pltpu.sync_copy(zeros_hbm, hist_v)
        pltpu.sync_copy(node_hbm.at[pl.ds(s * EPT_A, EPT_A)], node_all)
        pltpu.sync_copy(et_hbm.at[pl.ds(s * EPT_A, EPT_A)], et_all)

        def chunk_body(k, _):
            def group_body(j, _):
                o = k * CH_A + j * 16
                nd = node_all[pl.ds(o, 16)]
                tt = et_all[pl.ds(o, 16)]
                g = tt * N + nd
                g_c[pl.ds(j * 16, 16)] = g
                plsc.addupdate_scatter(hist_v, [g], ones)
                return 0

            lax.fori_loop(0, CH_A // 16, group_body, 0)
            pltpu.sync_copy(g_c, g_hbm.at[pl.ds(s * EPT_A + k * CH_A, CH_A)])
            return 0

        lax.fori_loop(0, NCH_A, chunk_body, 0)
        pltpu.sync_copy(hist_v, hist_out_hbm.at[s])

    @pl.when(c == 0)
    def _():
        tile_work(src_hbm, gsrc_hbm, histo_hbm)

    @pl.when(c == 1)
    def _():
        tile_work(dst_hbm, gdst_hbm, histi_hbm)


@functools.partial(
    pl.kernel,
    out_type=jax.ShapeDtypeStruct((E,), f32),
    mesh=_mesh,
    compiler_params=_sc_params,
    scratch_types=(
        pltpu.VMEM((RN,), f32),
        pltpu.VMEM((CH_W,), i32),
        pltpu.VMEM((CH_W,), f32),
    ),
)
def _edge_weight_kernel(normin_hbm, gdst_hbm, w_hbm, tab_v, gd_c, w_c):
    c = lax.axis_index("c")
    s = lax.axis_index("s")
    wid = c * NS + s
    pltpu.sync_copy(normin_hbm, tab_v)

    def chunk_body(k, _):
        off = wid * EPT + k * CH_W
        pltpu.sync_copy(gdst_hbm.at[pl.ds(off, CH_W)], gd_c)

        def group_body(j, _):
            g = gd_c[pl.ds(j * 16, 16)]
            w_c[pl.ds(j * 16, 16)] = plsc.load_gather(tab_v, [g])
            return 0

        lax.fori_loop(0, CH_W // 16, group_body, 0)
        pltpu.sync_copy(w_c, w_hbm.at[pl.ds(off, CH_W)])
        return 0

    lax.fori_loop(0, NCH_W, chunk_body, 0)


@functools.partial(
    pl.kernel,
    out_type=(
        jax.ShapeDtypeStruct((N, F), f32),
        jax.ShapeDtypeStruct((N, F), f32),
    ),
    mesh=_mesh,
    compiler_params=_sc_params,
    scratch_types=(
        pltpu.VMEM_SHARED((N, F), f32),
        pltpu.VMEM((CH_C, F), f32),
        pltpu.VMEM((CH_C, F), f32),
        pltpu.VMEM((NCH_C // 2, CH_C), i32),
        pltpu.VMEM((NCH_C // 2, CH_C), i32),
        pltpu.VMEM((NCH_C // 2, F), f32),
        pltpu.SemaphoreType.DMA,
        pltpu.SemaphoreType.DMA,
        pltpu.SemaphoreType.DMA,
        pltpu.SemaphoreType.DMA,
    ),
)
def _edge_pass_kernel(y_hbm, gsrc_hbm, dst_hbm, w_hbm, zeros_hbm,
                      p0_hbm, p1_hbm,
                      acc_sh, rows0, rows1, gs_all, ds_all, w_all,
                      sem0, sem1, ssem0, ssem1):
    c = lax.axis_index("c")
    s = lax.axis_index("s")
    wid = c * NS + s

    @pl.when(s < 10)
    def _():
        pltpu.sync_copy(zeros_hbm, acc_sh.at[pl.ds(s * 1000, 1000)])

    plsc.subcore_barrier()

    HC = NCH_C // 2

    def start_gather(k, rows, sem):
        pltpu.async_copy(y_hbm.at[gs_all.at[k]], rows, sem)

    def wait_gather(k, rows, sem):
        pltpu.make_async_copy(y_hbm.at[gs_all.at[k]], rows, sem).wait()

    _gdn = lax.GatherDimensionNumbers(
        offset_dims=(), collapsed_slice_dims=(0,), start_index_map=(0,))

    def _splat(w16, u):
        return lax.gather(w16, jnp.full((16, 1), u, dtype=i32), _gdn,
                          slice_sizes=(1,),
                          mode=lax.GatherScatterMode.PROMISE_IN_BOUNDS)

    def _scale_group(rows, w16, rb, nu):
        for u in range(nu):
            ws = _splat(w16, u)
            e = rb + u
            for t in range(8):
                sl = rows[e, pl.ds(t * 16, 16)]
                rows[e, pl.ds(t * 16, 16)] = sl * ws

    def scale(k, rows):
        def body(g, _):
            w16 = w_all[k, pl.ds(g * 16, 16)]
            _scale_group(rows, w16, g * 16, 16)
            return 0

        lax.fori_loop(0, CH_C // 16, body, 0)
        w16 = w_all[k, pl.ds(112, 16)]
        _scale_group(rows, w16, 112, CH_C - 112)

    def start_flush(k, rows, ssem):
        pltpu.async_copy(rows, acc_sh.at[ds_all.at[k]], ssem, add=True)

    def wait_flush(k, rows, ssem):
        pltpu.make_async_copy(rows, acc_sh.at[ds_all.at[k]], ssem).wait()

    def half_body(h, _):
        pltpu.sync_copy(gsrc_hbm.at[pl.ds(wid * NCH_C + h * HC, HC)], gs_all)
        pltpu.sync_copy(dst_hbm.at[pl.ds(wid * NCH_C + h * HC, HC)], ds_all)
        pltpu.sync_copy(w_hbm.at[pl.ds(wid * NCH_C + h * HC, HC)], w_all)

        start_gather(0, rows0, sem0)
        start_gather(1, rows1, sem1)

        def loop_body(i, _):
            k0 = i * 2
            wait_gather(k0, rows0, sem0)
            scale(k0, rows0)
            start_flush(k0, rows0, ssem0)
            wait_gather(k0 + 1, rows1, sem1)
            scale(k0 + 1, rows1)
            wait_flush(k0, rows0, ssem0)

            @pl.when(i < HC // 2 - 1)
            def _():
                start_gather(k0 + 2, rows0, sem0)

            start_flush(k0 + 1, rows1, ssem1)
            wait_flush(k0 + 1, rows1, ssem1)

            @pl.when(i < HC // 2 - 1)
            def _():
                start_gather(k0 + 3, rows1, sem1)

            return 0

        lax.fori_loop(0, HC // 2, loop_body, 0)
        return 0

    lax.fori_loop(0, 2, half_body, 0)

    plsc.subcore_barrier()
    sl = pl.ds(s * 1000, 1000)

    @pl.when((c == 0) & (s < 10))
    def _():
        pltpu.sync_copy(acc_sh.at[sl], p0_hbm.at[sl])

    @pl.when((c == 1) & (s < 10))
    def _():
        pltpu.sync_copy(acc_sh.at[sl], p1_hbm.at[sl])


def _mm1_body(x_ref, w_ref, histo_ref, histi_ref, y_ref, normo_ref, normi_ref):
    dego = jnp.sum(histo_ref[:, 0, 0, :], axis=0)
    degi = jnp.sum(histi_ref[:, 0, 0, :], axis=0)
    nrm = lax.rsqrt(jnp.maximum(dego, 1.0))
    normo_ref[0, 0] = nrm
    normi_ref[0, 0] = lax.rsqrt(jnp.maximum(degi, 1.0))
    y_ref[...] = jnp.dot(x_ref[...], w_ref[0],
                         preferred_element_type=f32) * nrm[:, None]


def _mm1(x, w, histo, histi):
    return pl.pallas_call(
        _mm1_body,
        grid=(R, NB),
        in_specs=[
            pl.BlockSpec((BN, F), lambda r, i: (i, 0)),
            pl.BlockSpec((1, F, F), lambda r, i: (r, 0, 0)),
            pl.BlockSpec((NS, 1, 1, BN), lambda r, i: (0, r * NB + i, 0, 0)),
            pl.BlockSpec((NS, 1, 1, BN), lambda r, i: (0, r * NB + i, 0, 0)),
        ],
        out_specs=(
            pl.BlockSpec((BN, F), lambda r, i: (r * NB + i, 0)),
            pl.BlockSpec((1, 1, BN), lambda r, i: (r * NB + i, 0, 0)),
            pl.BlockSpec((1, 1, BN), lambda r, i: (r * NB + i, 0, 0)),
        ),
        out_shape=(
            jax.ShapeDtypeStruct((RN, F), f32),
            jax.ShapeDtypeStruct((R * NB, 1, BN), f32),
            jax.ShapeDtypeStruct((R * NB, 1, BN), f32),
        ),
    )(x, w, histo, histi)


def _mm2_body(p0_ref, p1_ref, b1_ref, w_ref, nrm_ref, y_ref):
    h = p0_ref[...] + p1_ref[...] + jnp.sum(b1_ref[...], axis=0)[None, :]
    h = jnp.maximum(h, 0.0)
    y_ref[...] = jnp.dot(h, w_ref[0],
                         preferred_element_type=f32) * nrm_ref[0, 0][:, None]


def _mm2(p0, p1, b1, w2, normo):
    return pl.pallas_call(
        _mm2_body,
        grid=(R, NB),
        in_specs=[
            pl.BlockSpec((BN, F), lambda r, i: (i, 0)),
            pl.BlockSpec((BN, F), lambda r, i: (i, 0)),
            pl.BlockSpec((R, F), lambda r, i: (0, 0)),
            pl.BlockSpec((1, F, F), lambda r, i: (r, 0, 0)),
            pl.BlockSpec((1, 1, BN), lambda r, i: (r * NB + i, 0, 0)),
        ],
        out_specs=pl.BlockSpec((BN, F), lambda r, i: (r * NB + i, 0)),
        out_shape=jax.ShapeDtypeStruct((RN, F), f32),
    )(p0, p1, b1, w2, normo)


def _final_body(p0_ref, p1_ref, b2_ref, o_ref):
    o_ref[...] = (p0_ref[...] + p1_ref[...]
                  + jnp.sum(b2_ref[...], axis=0)[None, :])


def _final(p0, p1, b2):
    return pl.pallas_call(
        _final_body,
        grid=(NB,),
        in_specs=[
            pl.BlockSpec((BN, F), lambda i: (i, 0)),
            pl.BlockSpec((BN, F), lambda i: (i, 0)),
            pl.BlockSpec((R, F), lambda i: (0, 0)),
        ],
        out_specs=pl.BlockSpec((BN, F), lambda i: (i, 0)),
        out_shape=jax.ShapeDtypeStruct((N, F), f32),
    )(p0, p1, b2)


def kernel(features, edge_index, edge_type, W1, b1, W2, b2):
    src = edge_index[0]
    dst = edge_index[1]
    et = edge_type
    zeros1000 = jnp.zeros((1000, F), f32)
    zeros_rn = jnp.zeros((RN,), f32)

    histo, histi, gsrc, gdst = _hist_kernel(src, dst, et, zeros_rn)
    y1, normo, normi = _mm1(features, W1,
                            jnp.reshape(histo, (NS, R * NB, 1, BN)),
                            jnp.reshape(histi, (NS, R * NB, 1, BN)))
    w_e = _edge_weight_kernel(jnp.reshape(normi, (RN,)), gdst)
    gsrc2 = jnp.reshape(gsrc, (E // CH_C, CH_C))
    dst2 = jnp.reshape(dst, (E // CH_C, CH_C))
    w2d = jnp.pad(jnp.reshape(w_e, (E // CH_C, CH_C)),
                  ((0, 0), (0, F - CH_C)))

    p0, p1 = _edge_pass_kernel(y1, gsrc2, dst2, w2d, zeros1000)
    y2 = _mm2(p0, p1, b1, W2, normo)
    q0, q1 = _edge_pass_kernel(y2, gsrc2, dst2, w2d, zeros1000)
    return _final(q0, q1, b2)


# speedup vs baseline: 1.0481x; 1.0481x over previous
import functools

import jax
import jax.numpy as jnp
from jax import lax
from jax.experimental import pallas as pl
from jax.experimental.pallas import tpu as pltpu
from jax.experimental.pallas import tpu_sc as plsc

N = 10000
E = 320000
R = 8
F = 128
RN = R * N
HR = RN // F

NC = 2
NS = 16
NW = NC * NS

EPT_A = E // NS
CH_A = 2000
NCH_A = EPT_A // CH_A

EPT = E // NW
CH_W = 2000
NCH_W = EPT // CH_W
CH_C = 125
NCH_C = EPT // CH_C

BN = 400
NB = N // BN

_mesh = plsc.VectorSubcoreMesh(
    core_axis_name="c", subcore_axis_name="s", num_cores=NC, num_subcores=NS)
_sc_params = pltpu.CompilerParams(needs_layout_passes=False)

f32 = jnp.float32
i32 = jnp.int32


@functools.partial(
    pl.kernel,
    out_type=(
        jax.ShapeDtypeStruct((NS, RN), f32),
        jax.ShapeDtypeStruct((NS, RN), f32),
        jax.ShapeDtypeStruct((E,), i32),
        jax.ShapeDtypeStruct((E,), i32),
    ),
    mesh=_mesh,
    compiler_params=_sc_params,
    scratch_types=(
        pltpu.VMEM((RN,), f32),
        pltpu.VMEM((EPT_A,), i32),
        pltpu.VMEM((EPT_A,), i32),
        pltpu.VMEM((CH_A,), i32),
    ),
)
def _hist_kernel(src_hbm, dst_hbm, et_hbm, zeros_hbm,
                 histo_hbm, histi_hbm, gsrc_hbm, gdst_hbm,
                 hist_v, node_all, et_all, g_c):
    c = lax.axis_index("c")
    s = lax.axis_index("s")
    ones = jnp.ones((16,), f32)

    def tile_work(node_hbm, g_hbm, hist_out_hbm):
        pltpu.sync_copy(zeros_hbm, hist_v)
        pltpu.sync_copy(node_hbm.at[pl.ds(s * EPT_A, EPT_A)], node_all)
        pltpu.sync_copy(et_hbm.at[pl.ds(s * EPT_A, EPT_A)], et_all)

        def chunk_body(k, _):
            def group_body(j, _):
                o = k * CH_A + j * 16
                nd = node_all[pl.ds(o, 16)]
                tt = et_all[pl.ds(o, 16)]
                g = tt * N + nd
                g_c[pl.ds(j * 16, 16)] = g
                plsc.addupdate_scatter(hist_v, [g], ones)
                return 0

            lax.fori_loop(0, CH_A // 16, group_body, 0)
            pltpu.sync_copy(g_c, g_hbm.at[pl.ds(s * EPT_A + k * CH_A, CH_A)])
            return 0

        lax.fori_loop(0, NCH_A, chunk_body, 0)
        pltpu.sync_copy(hist_v, hist_out_hbm.at[s])

    @pl.when(c == 0)
    def _():
        tile_work(src_hbm, gsrc_hbm, histo_hbm)

    @pl.when(c == 1)
    def _():
        tile_work(dst_hbm, gdst_hbm, histi_hbm)


@functools.partial(
    pl.kernel,
    out_type=jax.ShapeDtypeStruct((E,), f32),
    mesh=_mesh,
    compiler_params=_sc_params,
    scratch_types=(
        pltpu.VMEM((RN,), f32),
        pltpu.VMEM((CH_W,), i32),
        pltpu.VMEM((CH_W,), f32),
    ),
)
def _edge_weight_kernel(normin_hbm, gdst_hbm, w_hbm, tab_v, gd_c, w_c):
    c = lax.axis_index("c")
    s = lax.axis_index("s")
    wid = c * NS + s
    pltpu.sync_copy(normin_hbm, tab_v)

    def chunk_body(k, _):
        off = wid * EPT + k * CH_W
        pltpu.sync_copy(gdst_hbm.at[pl.ds(off, CH_W)], gd_c)

        def group_body(j, _):
            g = gd_c[pl.ds(j * 16, 16)]
            w_c[pl.ds(j * 16, 16)] = plsc.load_gather(tab_v, [g])
            return 0

        lax.fori_loop(0, CH_W // 16, group_body, 0)
        pltpu.sync_copy(w_c, w_hbm.at[pl.ds(off, CH_W)])
        return 0

    lax.fori_loop(0, NCH_W, chunk_body, 0)


@functools.partial(
    pl.kernel,
    out_type=(
        jax.ShapeDtypeStruct((N, F), f32),
        jax.ShapeDtypeStruct((N, F), f32),
    ),
    mesh=_mesh,
    compiler_params=_sc_params,
    scratch_types=(
        pltpu.VMEM_SHARED((N, F), f32),
        pltpu.VMEM((CH_C, F), f32),
        pltpu.VMEM((CH_C, F), f32),
        pltpu.VMEM((NCH_C // 2, CH_C), i32),
        pltpu.VMEM((NCH_C // 2, CH_C), i32),
        pltpu.VMEM((NCH_C // 2, F), f32),
        pltpu.SemaphoreType.DMA,
        pltpu.SemaphoreType.DMA,
        pltpu.SemaphoreType.DMA,
        pltpu.SemaphoreType.DMA,
    ),
)
def _edge_pass_kernel(y_hbm, gsrc_hbm, dst_hbm, w_hbm, zeros_hbm,
                      p0_hbm, p1_hbm,
                      acc_sh, rows0, rows1, gs_all, ds_all, w_all,
                      sem0, sem1, ssem0, ssem1):
    c = lax.axis_index("c")
    s = lax.axis_index("s")
    wid = c * NS + s

    @pl.when(s < 10)
    def _():
        pltpu.sync_copy(zeros_hbm, acc_sh.at[pl.ds(s * 1000, 1000)])

    plsc.subcore_barrier()

    HC = NCH_C // 2

    def start_gather(k, rows, sem):
        pltpu.async_copy(y_hbm.at[gs_all.at[k]], rows, sem)

    def wait_gather(k, rows, sem):
        pltpu.make_async_copy(y_hbm.at[gs_all.at[k]], rows, sem).wait()

    _gdn = lax.GatherDimensionNumbers(
        offset_dims=(), collapsed_slice_dims=(0,), start_index_map=(0,))

    def _splat(w16, u):
        return lax.gather(w16, jnp.full((16, 1), u, dtype=i32), _gdn,
                          slice_sizes=(1,),
                          mode=lax.GatherScatterMode.PROMISE_IN_BOUNDS)

    def _scale_group(rows, w16, rb, nu):
        for u in range(nu):
            ws = _splat(w16, u)
            e = rb + u
            for t in range(8):
                sl = rows[e, pl.ds(t * 16, 16)]
                rows[e, pl.ds(t * 16, 16)] = sl * ws

    def scale(k, rows):
        def body(g, _):
            w16 = w_all[k, pl.ds(g * 16, 16)]
            _scale_group(rows, w16, g * 16, 16)
            return 0

        lax.fori_loop(0, CH_C // 16, body, 0)
        w16 = w_all[k, pl.ds(112, 16)]
        _scale_group(rows, w16, 112, CH_C - 112)

    def start_flush(k, rows, ssem):
        pltpu.async_copy(rows, acc_sh.at[ds_all.at[k]], ssem, add=True)

    def wait_flush(k, rows, ssem):
        pltpu.make_async_copy(rows, acc_sh.at[ds_all.at[k]], ssem).wait()

    def half_body(h, _):
        pltpu.sync_copy(gsrc_hbm.at[pl.ds(wid * NCH_C + h * HC, HC)], gs_all)
        pltpu.sync_copy(dst_hbm.at[pl.ds(wid * NCH_C + h * HC, HC)], ds_all)
        pltpu.sync_copy(w_hbm.at[pl.ds(wid * NCH_C + h * HC, HC)], w_all)

        start_gather(0, rows0, sem0)
        start_gather(1, rows1, sem1)

        def loop_body(i, _):
            k0 = i * 2
            wait_gather(k0, rows0, sem0)
            scale(k0, rows0)
            start_flush(k0, rows0, ssem0)
            wait_gather(k0 + 1, rows1, sem1)
            scale(k0 + 1, rows1)
            wait_flush(k0, rows0, ssem0)

            @pl.when(i < HC // 2 - 1)
            def _():
                start_gather(k0 + 2, rows0, sem0)

            start_flush(k0 + 1, rows1, ssem1)
            wait_flush(k0 + 1, rows1, ssem1)

            @pl.when(i < HC // 2 - 1)
            def _():
                start_gather(k0 + 3, rows1, sem1)

            return 0

        lax.fori_loop(0, HC // 2, loop_body, 0)
        return 0

    lax.fori_loop(0, 2, half_body, 0)

    plsc.subcore_barrier()
    sl = pl.ds(s * 1000, 1000)

    @pl.when((c == 0) & (s < 10))
    def _():
        pltpu.sync_copy(acc_sh.at[sl], p0_hbm.at[sl])

    @pl.when((c == 1) & (s < 10))
    def _():
        pltpu.sync_copy(acc_sh.at[sl], p1_hbm.at[sl])


def _deg_body(histo_ref, histi_ref, normo_ref, normi_ref):
    dego = jnp.sum(histo_ref[...], axis=0)
    degi = jnp.sum(histi_ref[...], axis=0)
    normo_ref[...] = lax.rsqrt(jnp.maximum(dego, 1.0))
    normi_ref[...] = lax.rsqrt(jnp.maximum(degi, 1.0))


def _degnorm(histo, histi):
    return pl.pallas_call(
        _deg_body,
        out_shape=(jax.ShapeDtypeStruct((HR, F), f32),
                   jax.ShapeDtypeStruct((HR, F), f32)),
    )(histo, histi)


def _mm1_body(x_ref, w_ref, nrm_ref, y_ref):
    y_ref[...] = jnp.dot(x_ref[...], w_ref[0],
                         preferred_element_type=f32) * nrm_ref[0, 0][:, None]


def _mm1(x, w, normo):
    return pl.pallas_call(
        _mm1_body,
        grid=(R, NB),
        in_specs=[
            pl.BlockSpec((BN, F), lambda r, i: (i, 0)),
            pl.BlockSpec((1, F, F), lambda r, i: (r, 0, 0)),
            pl.BlockSpec((1, 1, BN), lambda r, i: (r * NB + i, 0, 0)),
        ],
        out_specs=pl.BlockSpec((BN, F), lambda r, i: (r * NB + i, 0)),
        out_shape=jax.ShapeDtypeStruct((RN, F), f32),
    )(x, w, normo)


def _mm2_body(p0_ref, p1_ref, b1_ref, w_ref, nrm_ref, y_ref):
    h = p0_ref[...] + p1_ref[...] + jnp.sum(b1_ref[...], axis=0)[None, :]
    h = jnp.maximum(h, 0.0)
    y_ref[...] = jnp.dot(h, w_ref[0],
                         preferred_element_type=f32) * nrm_ref[0, 0][:, None]


def _mm2(p0, p1, b1, w2, normo):
    return pl.pallas_call(
        _mm2_body,
        grid=(R, NB),
        in_specs=[
            pl.BlockSpec((BN, F), lambda r, i: (i, 0)),
            pl.BlockSpec((BN, F), lambda r, i: (i, 0)),
            pl.BlockSpec((R, F), lambda r, i: (0, 0)),
            pl.BlockSpec((1, F, F), lambda r, i: (r, 0, 0)),
            pl.BlockSpec((1, 1, BN), lambda r, i: (r * NB + i, 0, 0)),
        ],
        out_specs=pl.BlockSpec((BN, F), lambda r, i: (r * NB + i, 0)),
        out_shape=jax.ShapeDtypeStruct((RN, F), f32),
    )(p0, p1, b1, w2, normo)


def _final_body(p0_ref, p1_ref, b2_ref, o_ref):
    o_ref[...] = (p0_ref[...] + p1_ref[...]
                  + jnp.sum(b2_ref[...], axis=0)[None, :])


def _final(p0, p1, b2):
    return pl.pallas_call(
        _final_body,
        grid=(NB,),
        in_specs=[
            pl.BlockSpec((BN, F), lambda i: (i, 0)),
            pl.BlockSpec((BN, F), lambda i: (i, 0)),
            pl.BlockSpec((R, F), lambda i: (0, 0)),
        ],
        out_specs=pl.BlockSpec((BN, F), lambda i: (i, 0)),
        out_shape=jax.ShapeDtypeStruct((N, F), f32),
    )(p0, p1, b2)


def kernel(features, edge_index, edge_type, W1, b1, W2, b2):
    src = edge_index[0]
    dst = edge_index[1]
    et = edge_type
    zeros1000 = jnp.zeros((1000, F), f32)
    zeros_rn = jnp.zeros((RN,), f32)

    histo, histi, gsrc, gdst = _hist_kernel(src, dst, et, zeros_rn)
    normo625, normi625 = _degnorm(jnp.reshape(histo, (NS, HR, F)),
                                  jnp.reshape(histi, (NS, HR, F)))
    w_e = _edge_weight_kernel(jnp.reshape(normi625, (RN,)), gdst)
    normo = jnp.reshape(normo625, (R * NB, 1, BN))
    gsrc2 = jnp.reshape(gsrc, (E // CH_C, CH_C))
    dst2 = jnp.reshape(dst, (E // CH_C, CH_C))
    w2d = jnp.pad(jnp.reshape(w_e, (E // CH_C, CH_C)),
                  ((0, 0), (0, F - CH_C)))

    y1 = _mm1(features, W1, normo)
    p0, p1 = _edge_pass_kernel(y1, gsrc2, dst2, w2d, zeros1000)
    y2 = _mm2(p0, p1, b1, W2, normo)
    q0, q1 = _edge_pass_kernel(y2, gsrc2, dst2, w2d, zeros1000)
    return _final(q0, q1, b2)


# speedup vs baseline: 1.3025x; 1.2427x over previous
import functools

import jax
import jax.numpy as jnp
from jax import lax
from jax.experimental import pallas as pl
from jax.experimental.pallas import tpu as pltpu
from jax.experimental.pallas import tpu_sc as plsc

N = 10000
E = 320000
R = 8
F = 128
RN = R * N
HR = RN // F

NC = 2
NS = 16
NW = NC * NS

EPT_A = E // NS
CH_A = 2000
NCH_A = EPT_A // CH_A

EPT = E // NW
CH_W = 2000
NCH_W = EPT // CH_W
CH_C = 125
NCH_C = EPT // CH_C

BN = 1000
NB = N // BN

_mesh = plsc.VectorSubcoreMesh(
    core_axis_name="c", subcore_axis_name="s", num_cores=NC, num_subcores=NS)
_sc_params = pltpu.CompilerParams(needs_layout_passes=False)

f32 = jnp.float32
i32 = jnp.int32


@functools.partial(
    pl.kernel,
    out_type=(
        jax.ShapeDtypeStruct((NS, RN), f32),
        jax.ShapeDtypeStruct((NS, RN), f32),
        jax.ShapeDtypeStruct((E,), i32),
        jax.ShapeDtypeStruct((E,), i32),
    ),
    mesh=_mesh,
    compiler_params=_sc_params,
    scratch_types=(
        pltpu.VMEM((RN,), f32),
        pltpu.VMEM((EPT_A,), i32),
        pltpu.VMEM((EPT_A,), i32),
        pltpu.VMEM((CH_A,), i32),
    ),
)
def _hist_kernel(src_hbm, dst_hbm, et_hbm, zeros_hbm,
                 histo_hbm, histi_hbm, gsrc_hbm, gdst_hbm,
                 hist_v, node_all, et_all, g_c):
    c = lax.axis_index("c")
    s = lax.axis_index("s")
    ones = jnp.ones((16,), f32)

    def tile_work(node_hbm, g_hbm, hist_out_hbm):
        pltpu.sync_copy(zeros_hbm, hist_v)
        pltpu.sync_copy(node_hbm.at[pl.ds(s * EPT_A, EPT_A)], node_all)
        pltpu.sync_copy(et_hbm.at[pl.ds(s * EPT_A, EPT_A)], et_all)

        def chunk_body(k, _):
            def group_body(j, _):
                o = k * CH_A + j * 16
                nd = node_all[pl.ds(o, 16)]
                tt = et_all[pl.ds(o, 16)]
                g = tt * N + nd
                g_c[pl.ds(j * 16, 16)] = g
                plsc.addupdate_scatter(hist_v, [g], ones)
                return 0

            lax.fori_loop(0, CH_A // 16, group_body, 0)
            pltpu.sync_copy(g_c, g_hbm.at[pl.ds(s * EPT_A + k * CH_A, CH_A)])
            return 0

        lax.fori_loop(0, NCH_A, chunk_body, 0)
        pltpu.sync_copy(hist_v, hist_out_hbm.at[s])

    @pl.when(c == 0)
    def _():
        tile_work(src_hbm, gsrc_hbm, histo_hbm)

    @pl.when(c == 1)
    def _():
        tile_work(dst_hbm, gdst_hbm, histi_hbm)


@functools.partial(
    pl.kernel,
    out_type=jax.ShapeDtypeStruct((E,), f32),
    mesh=_mesh,
    compiler_params=_sc_params,
    scratch_types=(
        pltpu.VMEM((RN,), f32),
        pltpu.VMEM((CH_W,), i32),
        pltpu.VMEM((CH_W,), f32),
    ),
)
def _edge_weight_kernel(normin_hbm, gdst_hbm, w_hbm, tab_v, gd_c, w_c):
    c = lax.axis_index("c")
    s = lax.axis_index("s")
    wid = c * NS + s
    pltpu.sync_copy(normin_hbm, tab_v)

    def chunk_body(k, _):
        off = wid * EPT + k * CH_W
        pltpu.sync_copy(gdst_hbm.at[pl.ds(off, CH_W)], gd_c)

        def group_body(j, _):
            g = gd_c[pl.ds(j * 16, 16)]
            w_c[pl.ds(j * 16, 16)] = plsc.load_gather(tab_v, [g])
            return 0

        lax.fori_loop(0, CH_W // 16, group_body, 0)
        pltpu.sync_copy(w_c, w_hbm.at[pl.ds(off, CH_W)])
        return 0

    lax.fori_loop(0, NCH_W, chunk_body, 0)


@functools.partial(
    pl.kernel,
    out_type=(
        jax.ShapeDtypeStruct((N, F), f32),
        jax.ShapeDtypeStruct((N, F), f32),
    ),
    mesh=_mesh,
    compiler_params=_sc_params,
    scratch_types=(
        pltpu.VMEM_SHARED((N, F), f32),
        pltpu.VMEM((CH_C, F), f32),
        pltpu.VMEM((CH_C, F), f32),
        pltpu.VMEM((NCH_C // 2, CH_C), i32),
        pltpu.VMEM((NCH_C // 2, CH_C), i32),
        pltpu.VMEM((NCH_C // 2, F), f32),
        pltpu.SemaphoreType.DMA,
        pltpu.SemaphoreType.DMA,
        pltpu.SemaphoreType.DMA,
        pltpu.SemaphoreType.DMA,
    ),
)
def _edge_pass_kernel(y_hbm, gsrc_hbm, dst_hbm, w_hbm, zeros_hbm,
                      p0_hbm, p1_hbm,
                      acc_sh, rows0, rows1, gs_all, ds_all, w_all,
                      sem0, sem1, ssem0, ssem1):
    c = lax.axis_index("c")
    s = lax.axis_index("s")
    wid = c * NS + s

    @pl.when(s < 10)
    def _():
        pltpu.sync_copy(zeros_hbm, acc_sh.at[pl.ds(s * 1000, 1000)])

    plsc.subcore_barrier()

    HC = NCH_C // 2

    def start_gather(k, rows, sem):
        pltpu.async_copy(y_hbm.at[gs_all.at[k]], rows, sem)

    def wait_gather(k, rows, sem):
        pltpu.make_async_copy(y_hbm.at[gs_all.at[k]], rows, sem).wait()

    _gdn = lax.GatherDimensionNumbers(
        offset_dims=(), collapsed_slice_dims=(0,), start_index_map=(0,))

    def _splat(w16, u):
        return lax.gather(w16, jnp.full((16, 1), u, dtype=i32), _gdn,
                          slice_sizes=(1,),
                          mode=lax.GatherScatterMode.PROMISE_IN_BOUNDS)

    def _scale_group(rows, w16, rb, nu):
        for u in range(nu):
            ws = _splat(w16, u)
            e = rb + u
            for t in range(8):
                sl = rows[e, pl.ds(t * 16, 16)]
                rows[e, pl.ds(t * 16, 16)] = sl * ws

    def scale(k, rows):
        def body(g, _):
            w16 = w_all[k, pl.ds(g * 16, 16)]
            _scale_group(rows, w16, g * 16, 16)
            return 0

        lax.fori_loop(0, CH_C // 16, body, 0)
        w16 = w_all[k, pl.ds(112, 16)]
        _scale_group(rows, w16, 112, CH_C - 112)

    def start_flush(k, rows, ssem):
        pltpu.async_copy(rows, acc_sh.at[ds_all.at[k]], ssem, add=True)

    def wait_flush(k, rows, ssem):
        pltpu.make_async_copy(rows, acc_sh.at[ds_all.at[k]], ssem).wait()

    def half_body(h, _):
        pltpu.sync_copy(gsrc_hbm.at[pl.ds(wid * NCH_C + h * HC, HC)], gs_all)
        pltpu.sync_copy(dst_hbm.at[pl.ds(wid * NCH_C + h * HC, HC)], ds_all)
        pltpu.sync_copy(w_hbm.at[pl.ds(wid * NCH_C + h * HC, HC)], w_all)

        start_gather(0, rows0, sem0)
        start_gather(1, rows1, sem1)

        def loop_body(i, _):
            k0 = i * 2
            wait_gather(k0, rows0, sem0)
            scale(k0, rows0)
            start_flush(k0, rows0, ssem0)
            wait_gather(k0 + 1, rows1, sem1)
            scale(k0 + 1, rows1)
            wait_flush(k0, rows0, ssem0)

            @pl.when(i < HC // 2 - 1)
            def _():
                start_gather(k0 + 2, rows0, sem0)

            start_flush(k0 + 1, rows1, ssem1)
            wait_flush(k0 + 1, rows1, ssem1)

            @pl.when(i < HC // 2 - 1)
            def _():
                start_gather(k0 + 3, rows1, sem1)

            return 0

        lax.fori_loop(0, HC // 2, loop_body, 0)
        return 0

    lax.fori_loop(0, 2, half_body, 0)

    plsc.subcore_barrier()
    sl = pl.ds(s * 1000, 1000)

    @pl.when((c == 0) & (s < 10))
    def _():
        pltpu.sync_copy(acc_sh.at[sl], p0_hbm.at[sl])

    @pl.when((c == 1) & (s < 10))
    def _():
        pltpu.sync_copy(acc_sh.at[sl], p1_hbm.at[sl])


def _deg_body(histo_ref, histi_ref, normo_ref, normi_ref):
    dego = jnp.sum(histo_ref[...], axis=0)
    degi = jnp.sum(histi_ref[...], axis=0)
    normo_ref[...] = lax.rsqrt(jnp.maximum(dego, 1.0))
    normi_ref[...] = lax.rsqrt(jnp.maximum(degi, 1.0))


def _degnorm(histo, histi):
    return pl.pallas_call(
        _deg_body,
        out_shape=(jax.ShapeDtypeStruct((HR, F), f32),
                   jax.ShapeDtypeStruct((HR, F), f32)),
    )(histo, histi)


def _mm1_body(x_ref, w_ref, nrm_ref, y_ref):
    y_ref[...] = jnp.dot(x_ref[...], w_ref[0],
                         preferred_element_type=f32) * nrm_ref[0, 0][:, None]


def _mm1(x, w, normo):
    return pl.pallas_call(
        _mm1_body,
        grid=(R, NB),
        in_specs=[
            pl.BlockSpec((BN, F), lambda r, i: (i, 0)),
            pl.BlockSpec((1, F, F), lambda r, i: (r, 0, 0)),
            pl.BlockSpec((1, 1, BN), lambda r, i: (r * NB + i, 0, 0)),
        ],
        out_specs=pl.BlockSpec((BN, F), lambda r, i: (r * NB + i, 0)),
        out_shape=jax.ShapeDtypeStruct((RN, F), f32),
    )(x, w, normo)


def _mm2_body(p0_ref, p1_ref, b1_ref, w_ref, nrm_ref, y_ref):
    h = p0_ref[...] + p1_ref[...] + jnp.sum(b1_ref[...], axis=0)[None, :]
    h = jnp.maximum(h, 0.0)
    y_ref[...] = jnp.dot(h, w_ref[0],
                         preferred_element_type=f32) * nrm_ref[0, 0][:, None]


def _mm2(p0, p1, b1, w2, normo):
    return pl.pallas_call(
        _mm2_body,
        grid=(R, NB),
        in_specs=[
            pl.BlockSpec((BN, F), lambda r, i: (i, 0)),
            pl.BlockSpec((BN, F), lambda r, i: (i, 0)),
            pl.BlockSpec((R, F), lambda r, i: (0, 0)),
            pl.BlockSpec((1, F, F), lambda r, i: (r, 0, 0)),
            pl.BlockSpec((1, 1, BN), lambda r, i: (r * NB + i, 0, 0)),
        ],
        out_specs=pl.BlockSpec((BN, F), lambda r, i: (r * NB + i, 0)),
        out_shape=jax.ShapeDtypeStruct((RN, F), f32),
    )(p0, p1, b1, w2, normo)


def _final_body(p0_ref, p1_ref, b2_ref, o_ref):
    o_ref[...] = (p0_ref[...] + p1_ref[...]
                  + jnp.sum(b2_ref[...], axis=0)[None, :])


def _final(p0, p1, b2):
    return pl.pallas_call(
        _final_body,
        grid=(NB,),
        in_specs=[
            pl.BlockSpec((BN, F), lambda i: (i, 0)),
            pl.BlockSpec((BN, F), lambda i: (i, 0)),
            pl.BlockSpec((R, F), lambda i: (0, 0)),
        ],
        out_specs=pl.BlockSpec((BN, F), lambda i: (i, 0)),
        out_shape=jax.ShapeDtypeStruct((N, F), f32),
    )(p0, p1, b2)


def kernel(features, edge_index, edge_type, W1, b1, W2, b2):
    src = edge_index[0]
    dst = edge_index[1]
    et = edge_type
    zeros1000 = jnp.zeros((1000, F), f32)
    zeros_rn = jnp.zeros((RN,), f32)

    histo, histi, gsrc, gdst = _hist_kernel(src, dst, et, zeros_rn)
    normo625, normi625 = _degnorm(jnp.reshape(histo, (NS, HR, F)),
                                  jnp.reshape(histi, (NS, HR, F)))
    w_e = _edge_weight_kernel(jnp.reshape(normi625, (RN,)), gdst)
    normo = jnp.reshape(normo625, (R * NB, 1, BN))
    gsrc2 = jnp.reshape(gsrc, (E // CH_C, CH_C))
    dst2 = jnp.reshape(dst, (E // CH_C, CH_C))
    w2d = jnp.pad(jnp.reshape(w_e, (E // CH_C, CH_C)),
                  ((0, 0), (0, F - CH_C)))

    y1 = _mm1(features, W1, normo)
    p0, p1 = _edge_pass_kernel(y1, gsrc2, dst2, w2d, zeros1000)
    y2 = _mm2(p0, p1, b1, W2, normo)
    q0, q1 = _edge_pass_kernel(y2, gsrc2, dst2, w2d, zeros1000)
    return _final(q0, q1, b2)


# speedup vs baseline: 1.4321x; 1.0995x over previous
import functools

import jax
import jax.numpy as jnp
from jax import lax
from jax.experimental import pallas as pl
from jax.experimental.pallas import tpu as pltpu
from jax.experimental.pallas import tpu_sc as plsc

N = 10000
E = 320000
R = 8
F = 128
RN = R * N
HR = RN // F

NC = 2
NS = 16
NW = NC * NS

EPT_A = E // NS
CH_A = 2000
NCH_A = EPT_A // CH_A

EPT = E // NW
CH_W = 2000
NCH_W = EPT // CH_W
CH_C = 125
NCH_C = EPT // CH_C

BN = 2000
NB = N // BN

_mesh = plsc.VectorSubcoreMesh(
    core_axis_name="c", subcore_axis_name="s", num_cores=NC, num_subcores=NS)
_sc_params = pltpu.CompilerParams(needs_layout_passes=False)

f32 = jnp.float32
i32 = jnp.int32


@functools.partial(
    pl.kernel,
    out_type=(
        jax.ShapeDtypeStruct((NS, RN), f32),
        jax.ShapeDtypeStruct((NS, RN), f32),
        jax.ShapeDtypeStruct((E,), i32),
        jax.ShapeDtypeStruct((E,), i32),
    ),
    mesh=_mesh,
    compiler_params=_sc_params,
    scratch_types=(
        pltpu.VMEM((RN,), f32),
        pltpu.VMEM((EPT_A,), i32),
        pltpu.VMEM((EPT_A,), i32),
        pltpu.VMEM((CH_A,), i32),
    ),
)
def _hist_kernel(src_hbm, dst_hbm, et_hbm, zeros_hbm,
                 histo_hbm, histi_hbm, gsrc_hbm, gdst_hbm,
                 hist_v, node_all, et_all, g_c):
    c = lax.axis_index("c")
    s = lax.axis_index("s")
    ones = jnp.ones((16,), f32)

    def tile_work(node_hbm, g_hbm, hist_out_hbm):
        pltpu.sync_copy(zeros_hbm, hist_v)
        pltpu.sync_copy(node_hbm.at[pl.ds(s * EPT_A, EPT_A)], node_all)
        pltpu.sync_copy(et_hbm.at[pl.ds(s * EPT_A, EPT_A)], et_all)

        def chunk_body(k, _):
            def group_body(j, _):
                o = k * CH_A + j * 16
                nd = node_all[pl.ds(o, 16)]
                tt = et_all[pl.ds(o, 16)]
                g = tt * N + nd
                g_c[pl.ds(j * 16, 16)] = g
                plsc.addupdate_scatter(hist_v, [g], ones)
                return 0

            lax.fori_loop(0, CH_A // 16, group_body, 0)
            pltpu.sync_copy(g_c, g_hbm.at[pl.ds(s * EPT_A + k * CH_A, CH_A)])
            return 0

        lax.fori_loop(0, NCH_A, chunk_body, 0)
        pltpu.sync_copy(hist_v, hist_out_hbm.at[s])

    @pl.when(c == 0)
    def _():
        tile_work(src_hbm, gsrc_hbm, histo_hbm)

    @pl.when(c == 1)
    def _():
        tile_work(dst_hbm, gdst_hbm, histi_hbm)


@functools.partial(
    pl.kernel,
    out_type=jax.ShapeDtypeStruct((E,), f32),
    mesh=_mesh,
    compiler_params=_sc_params,
    scratch_types=(
        pltpu.VMEM((RN,), f32),
        pltpu.VMEM((CH_W,), i32),
        pltpu.VMEM((CH_W,), f32),
    ),
)
def _edge_weight_kernel(normin_hbm, gdst_hbm, w_hbm, tab_v, gd_c, w_c):
    c = lax.axis_index("c")
    s = lax.axis_index("s")
    wid = c * NS + s
    pltpu.sync_copy(normin_hbm, tab_v)

    def chunk_body(k, _):
        off = wid * EPT + k * CH_W
        pltpu.sync_copy(gdst_hbm.at[pl.ds(off, CH_W)], gd_c)

        def group_body(j, _):
            g = gd_c[pl.ds(j * 16, 16)]
            w_c[pl.ds(j * 16, 16)] = plsc.load_gather(tab_v, [g])
            return 0

        lax.fori_loop(0, CH_W // 16, group_body, 0)
        pltpu.sync_copy(w_c, w_hbm.at[pl.ds(off, CH_W)])
        return 0

    lax.fori_loop(0, NCH_W, chunk_body, 0)


@functools.partial(
    pl.kernel,
    out_type=(
        jax.ShapeDtypeStruct((N, F), f32),
        jax.ShapeDtypeStruct((N, F), f32),
    ),
    mesh=_mesh,
    compiler_params=_sc_params,
    scratch_types=(
        pltpu.VMEM_SHARED((N, F), f32),
        pltpu.VMEM((CH_C, F), f32),
        pltpu.VMEM((CH_C, F), f32),
        pltpu.VMEM((NCH_C // 2, CH_C), i32),
        pltpu.VMEM((NCH_C // 2, CH_C), i32),
        pltpu.VMEM((NCH_C // 2, F), f32),
        pltpu.SemaphoreType.DMA,
        pltpu.SemaphoreType.DMA,
        pltpu.SemaphoreType.DMA,
        pltpu.SemaphoreType.DMA,
    ),
)
def _edge_pass_kernel(y_hbm, gsrc_hbm, dst_hbm, w_hbm, zeros_hbm,
                      p0_hbm, p1_hbm,
                      acc_sh, rows0, rows1, gs_all, ds_all, w_all,
                      sem0, sem1, ssem0, ssem1):
    c = lax.axis_index("c")
    s = lax.axis_index("s")
    wid = c * NS + s

    @pl.when(s < 10)
    def _():
        pltpu.sync_copy(zeros_hbm, acc_sh.at[pl.ds(s * 1000, 1000)])

    plsc.subcore_barrier()

    HC = NCH_C // 2

    def start_gather(k, rows, sem):
        pltpu.async_copy(y_hbm.at[gs_all.at[k]], rows, sem)

    def wait_gather(k, rows, sem):
        pltpu.make_async_copy(y_hbm.at[gs_all.at[k]], rows, sem).wait()

    _gdn = lax.GatherDimensionNumbers(
        offset_dims=(), collapsed_slice_dims=(0,), start_index_map=(0,))

    def _splat(w16, u):
        return lax.gather(w16, jnp.full((16, 1), u, dtype=i32), _gdn,
                          slice_sizes=(1,),
                          mode=lax.GatherScatterMode.PROMISE_IN_BOUNDS)

    def _scale_group(rows, w16, rb, nu):
        for u in range(nu):
            ws = _splat(w16, u)
            e = rb + u
            for t in range(8):
                sl = rows[e, pl.ds(t * 16, 16)]
                rows[e, pl.ds(t * 16, 16)] = sl * ws

    def scale(k, rows):
        def body(g, _):
            w16 = w_all[k, pl.ds(g * 16, 16)]
            _scale_group(rows, w16, g * 16, 16)
            return 0

        lax.fori_loop(0, CH_C // 16, body, 0)
        w16 = w_all[k, pl.ds(112, 16)]
        _scale_group(rows, w16, 112, CH_C - 112)

    def start_flush(k, rows, ssem):
        pltpu.async_copy(rows, acc_sh.at[ds_all.at[k]], ssem, add=True)

    def wait_flush(k, rows, ssem):
        pltpu.make_async_copy(rows, acc_sh.at[ds_all.at[k]], ssem).wait()

    def half_body(h, _):
        pltpu.sync_copy(gsrc_hbm.at[pl.ds(wid * NCH_C + h * HC, HC)], gs_all)
        pltpu.sync_copy(dst_hbm.at[pl.ds(wid * NCH_C + h * HC, HC)], ds_all)
        pltpu.sync_copy(w_hbm.at[pl.ds(wid * NCH_C + h * HC, HC)], w_all)

        start_gather(0, rows0, sem0)
        start_gather(1, rows1, sem1)

        def loop_body(i, _):
            k0 = i * 2
            wait_gather(k0, rows0, sem0)
            scale(k0, rows0)
            start_flush(k0, rows0, ssem0)
            wait_gather(k0 + 1, rows1, sem1)
            scale(k0 + 1, rows1)
            wait_flush(k0, rows0, ssem0)

            @pl.when(i < HC // 2 - 1)
            def _():
                start_gather(k0 + 2, rows0, sem0)

            start_flush(k0 + 1, rows1, ssem1)
            wait_flush(k0 + 1, rows1, ssem1)

            @pl.when(i < HC // 2 - 1)
            def _():
                start_gather(k0 + 3, rows1, sem1)

            return 0

        lax.fori_loop(0, HC // 2, loop_body, 0)
        return 0

    lax.fori_loop(0, 2, half_body, 0)

    plsc.subcore_barrier()
    sl = pl.ds(s * 1000, 1000)

    @pl.when((c == 0) & (s < 10))
    def _():
        pltpu.sync_copy(acc_sh.at[sl], p0_hbm.at[sl])

    @pl.when((c == 1) & (s < 10))
    def _():
        pltpu.sync_copy(acc_sh.at[sl], p1_hbm.at[sl])


def _deg_body(histo_ref, histi_ref, normo_ref, normi_ref):
    dego = jnp.sum(histo_ref[...], axis=0)
    degi = jnp.sum(histi_ref[...], axis=0)
    normo_ref[...] = lax.rsqrt(jnp.maximum(dego, 1.0))
    normi_ref[...] = lax.rsqrt(jnp.maximum(degi, 1.0))


def _degnorm(histo, histi):
    return pl.pallas_call(
        _deg_body,
        out_shape=(jax.ShapeDtypeStruct((HR, F), f32),
                   jax.ShapeDtypeStruct((HR, F), f32)),
    )(histo, histi)


def _mm1_body(x_ref, w_ref, nrm_ref, y_ref):
    y_ref[...] = jnp.dot(x_ref[...], w_ref[0],
                         preferred_element_type=f32) * nrm_ref[0, 0][:, None]


def _mm1(x, w, normo):
    return pl.pallas_call(
        _mm1_body,
        grid=(R, NB),
        in_specs=[
            pl.BlockSpec((BN, F), lambda r, i: (i, 0)),
            pl.BlockSpec((1, F, F), lambda r, i: (r, 0, 0)),
            pl.BlockSpec((1, 1, BN), lambda r, i: (r * NB + i, 0, 0)),
        ],
        out_specs=pl.BlockSpec((BN, F), lambda r, i: (r * NB + i, 0)),
        out_shape=jax.ShapeDtypeStruct((RN, F), f32),
    )(x, w, normo)


def _mm2_body(p0_ref, p1_ref, b1_ref, w_ref, nrm_ref, y_ref):
    h = p0_ref[...] + p1_ref[...] + jnp.sum(b1_ref[...], axis=0)[None, :]
    h = jnp.maximum(h, 0.0)
    y_ref[...] = jnp.dot(h, w_ref[0],
                         preferred_element_type=f32) * nrm_ref[0, 0][:, None]


def _mm2(p0, p1, b1, w2, normo):
    return pl.pallas_call(
        _mm2_body,
        grid=(R, NB),
        in_specs=[
            pl.BlockSpec((BN, F), lambda r, i: (i, 0)),
            pl.BlockSpec((BN, F), lambda r, i: (i, 0)),
            pl.BlockSpec((R, F), lambda r, i: (0, 0)),
            pl.BlockSpec((1, F, F), lambda r, i: (r, 0, 0)),
            pl.BlockSpec((1, 1, BN), lambda r, i: (r * NB + i, 0, 0)),
        ],
        out_specs=pl.BlockSpec((BN, F), lambda r, i: (r * NB + i, 0)),
        out_shape=jax.ShapeDtypeStruct((RN, F), f32),
    )(p0, p1, b1, w2, normo)


def _final_body(p0_ref, p1_ref, b2_ref, o_ref):
    o_ref[...] = (p0_ref[...] + p1_ref[...]
                  + jnp.sum(b2_ref[...], axis=0)[None, :])


def _final(p0, p1, b2):
    return pl.pallas_call(
        _final_body,
        grid=(NB,),
        in_specs=[
            pl.BlockSpec((BN, F), lambda i: (i, 0)),
            pl.BlockSpec((BN, F), lambda i: (i, 0)),
            pl.BlockSpec((R, F), lambda i: (0, 0)),
        ],
        out_specs=pl.BlockSpec((BN, F), lambda i: (i, 0)),
        out_shape=jax.ShapeDtypeStruct((N, F), f32),
    )(p0, p1, b2)


def kernel(features, edge_index, edge_type, W1, b1, W2, b2):
    src = edge_index[0]
    dst = edge_index[1]
    et = edge_type
    zeros1000 = jnp.zeros((1000, F), f32)
    zeros_rn = jnp.zeros((RN,), f32)

    histo, histi, gsrc, gdst = _hist_kernel(src, dst, et, zeros_rn)
    normo625, normi625 = _degnorm(jnp.reshape(histo, (NS, HR, F)),
                                  jnp.reshape(histi, (NS, HR, F)))
    w_e = _edge_weight_kernel(jnp.reshape(normi625, (RN,)), gdst)
    normo = jnp.reshape(normo625, (R * NB, 1, BN))
    gsrc2 = jnp.reshape(gsrc, (E // CH_C, CH_C))
    dst2 = jnp.reshape(dst, (E // CH_C, CH_C))
    w2d = jnp.pad(jnp.reshape(w_e, (E // CH_C, CH_C)),
                  ((0, 0), (0, F - CH_C)))

    y1 = _mm1(features, W1, normo)
    p0, p1 = _edge_pass_kernel(y1, gsrc2, dst2, w2d, zeros1000)
    y2 = _mm2(p0, p1, b1, W2, normo)
    q0, q1 = _edge_pass_kernel(y2, gsrc2, dst2, w2d, zeros1000)
    return _final(q0, q1, b2)


# speedup vs baseline: 1.6223x; 1.1328x over previous
import functools

import jax
import jax.numpy as jnp
from jax import lax
from jax.experimental import pallas as pl
from jax.experimental.pallas import tpu as pltpu
from jax.experimental.pallas import tpu_sc as plsc

N = 10000
E = 320000
R = 8
F = 128
RN = R * N
HR = RN // F

NC = 2
NS = 16
NW = NC * NS

EPT_A = E // NS
CH_A = 2000
NCH_A = EPT_A // CH_A

EPT = E // NW
CH_W = 2000
NCH_W = EPT // CH_W
CH_C = 125
NCH_C = EPT // CH_C

BN = 10000
NB = N // BN

_mesh = plsc.VectorSubcoreMesh(
    core_axis_name="c", subcore_axis_name="s", num_cores=NC, num_subcores=NS)
_sc_params = pltpu.CompilerParams(needs_layout_passes=False)

f32 = jnp.float32
i32 = jnp.int32


@functools.partial(
    pl.kernel,
    out_type=(
        jax.ShapeDtypeStruct((NS, RN), f32),
        jax.ShapeDtypeStruct((NS, RN), f32),
        jax.ShapeDtypeStruct((E,), i32),
        jax.ShapeDtypeStruct((E,), i32),
    ),
    mesh=_mesh,
    compiler_params=_sc_params,
    scratch_types=(
        pltpu.VMEM((RN,), f32),
        pltpu.VMEM((EPT_A,), i32),
        pltpu.VMEM((EPT_A,), i32),
        pltpu.VMEM((CH_A,), i32),
    ),
)
def _hist_kernel(src_hbm, dst_hbm, et_hbm, zeros_hbm,
                 histo_hbm, histi_hbm, gsrc_hbm, gdst_hbm,
                 hist_v, node_all, et_all, g_c):
    c = lax.axis_index("c")
    s = lax.axis_index("s")
    ones = jnp.ones((16,), f32)

    def tile_work(node_hbm, g_hbm, hist_out_hbm):
        pltpu.sync_copy(zeros_hbm, hist_v)
        pltpu.sync_copy(node_hbm.at[pl.ds(s * EPT_A, EPT_A)], node_all)
        pltpu.sync_copy(et_hbm.at[pl.ds(s * EPT_A, EPT_A)], et_all)

        def chunk_body(k, _):
            def group_body(j, _):
                o = k * CH_A + j * 16
                nd = node_all[pl.ds(o, 16)]
                tt = et_all[pl.ds(o, 16)]
                g = tt * N + nd
                g_c[pl.ds(j * 16, 16)] = g
                plsc.addupdate_scatter(hist_v, [g], ones)
                return 0

            lax.fori_loop(0, CH_A // 16, group_body, 0)
            pltpu.sync_copy(g_c, g_hbm.at[pl.ds(s * EPT_A + k * CH_A, CH_A)])
            return 0

        lax.fori_loop(0, NCH_A, chunk_body, 0)
        pltpu.sync_copy(hist_v, hist_out_hbm.at[s])

    @pl.when(c == 0)
    def _():
        tile_work(src_hbm, gsrc_hbm, histo_hbm)

    @pl.when(c == 1)
    def _():
        tile_work(dst_hbm, gdst_hbm, histi_hbm)


@functools.partial(
    pl.kernel,
    out_type=jax.ShapeDtypeStruct((E,), f32),
    mesh=_mesh,
    compiler_params=_sc_params,
    scratch_types=(
        pltpu.VMEM((RN,), f32),
        pltpu.VMEM((CH_W,), i32),
        pltpu.VMEM((CH_W,), f32),
    ),
)
def _edge_weight_kernel(normin_hbm, gdst_hbm, w_hbm, tab_v, gd_c, w_c):
    c = lax.axis_index("c")
    s = lax.axis_index("s")
    wid = c * NS + s
    pltpu.sync_copy(normin_hbm, tab_v)

    def chunk_body(k, _):
        off = wid * EPT + k * CH_W
        pltpu.sync_copy(gdst_hbm.at[pl.ds(off, CH_W)], gd_c)

        def group_body(j, _):
            g = gd_c[pl.ds(j * 16, 16)]
            w_c[pl.ds(j * 16, 16)] = plsc.load_gather(tab_v, [g])
            return 0

        lax.fori_loop(0, CH_W // 16, group_body, 0)
        pltpu.sync_copy(w_c, w_hbm.at[pl.ds(off, CH_W)])
        return 0

    lax.fori_loop(0, NCH_W, chunk_body, 0)


@functools.partial(
    pl.kernel,
    out_type=(
        jax.ShapeDtypeStruct((N, F), f32),
        jax.ShapeDtypeStruct((N, F), f32),
    ),
    mesh=_mesh,
    compiler_params=_sc_params,
    scratch_types=(
        pltpu.VMEM_SHARED((N, F), f32),
        pltpu.VMEM((CH_C, F), f32),
        pltpu.VMEM((CH_C, F), f32),
        pltpu.VMEM((NCH_C // 2, CH_C), i32),
        pltpu.VMEM((NCH_C // 2, CH_C), i32),
        pltpu.VMEM((NCH_C // 2, F), f32),
        pltpu.SemaphoreType.DMA,
        pltpu.SemaphoreType.DMA,
        pltpu.SemaphoreType.DMA,
        pltpu.SemaphoreType.DMA,
    ),
)
def _edge_pass_kernel(y_hbm, gsrc_hbm, dst_hbm, w_hbm, zeros_hbm,
                      p0_hbm, p1_hbm,
                      acc_sh, rows0, rows1, gs_all, ds_all, w_all,
                      sem0, sem1, ssem0, ssem1):
    c = lax.axis_index("c")
    s = lax.axis_index("s")
    wid = c * NS + s

    @pl.when(s < 10)
    def _():
        pltpu.sync_copy(zeros_hbm, acc_sh.at[pl.ds(s * 1000, 1000)])

    plsc.subcore_barrier()

    HC = NCH_C // 2

    def start_gather(k, rows, sem):
        pltpu.async_copy(y_hbm.at[gs_all.at[k]], rows, sem)

    def wait_gather(k, rows, sem):
        pltpu.make_async_copy(y_hbm.at[gs_all.at[k]], rows, sem).wait()

    _gdn = lax.GatherDimensionNumbers(
        offset_dims=(), collapsed_slice_dims=(0,), start_index_map=(0,))

    def _splat(w16, u):
        return lax.gather(w16, jnp.full((16, 1), u, dtype=i32), _gdn,
                          slice_sizes=(1,),
                          mode=lax.GatherScatterMode.PROMISE_IN_BOUNDS)

    def _scale_group(rows, w16, rb, nu):
        for u in range(nu):
            ws = _splat(w16, u)
            e = rb + u
            for t in range(8):
                sl = rows[e, pl.ds(t * 16, 16)]
                rows[e, pl.ds(t * 16, 16)] = sl * ws

    def scale(k, rows):
        def body(g, _):
            w16 = w_all[k, pl.ds(g * 16, 16)]
            _scale_group(rows, w16, g * 16, 16)
            return 0

        lax.fori_loop(0, CH_C // 16, body, 0)
        w16 = w_all[k, pl.ds(112, 16)]
        _scale_group(rows, w16, 112, CH_C - 112)

    def start_flush(k, rows, ssem):
        pltpu.async_copy(rows, acc_sh.at[ds_all.at[k]], ssem, add=True)

    def wait_flush(k, rows, ssem):
        pltpu.make_async_copy(rows, acc_sh.at[ds_all.at[k]], ssem).wait()

    def half_body(h, _):
        pltpu.sync_copy(gsrc_hbm.at[pl.ds(wid * NCH_C + h * HC, HC)], gs_all)
        pltpu.sync_copy(dst_hbm.at[pl.ds(wid * NCH_C + h * HC, HC)], ds_all)
        pltpu.sync_copy(w_hbm.at[pl.ds(wid * NCH_C + h * HC, HC)], w_all)

        start_gather(0, rows0, sem0)
        start_gather(1, rows1, sem1)

        def loop_body(i, _):
            k0 = i * 2
            wait_gather(k0, rows0, sem0)
            scale(k0, rows0)
            start_flush(k0, rows0, ssem0)
            wait_gather(k0 + 1, rows1, sem1)
            scale(k0 + 1, rows1)
            wait_flush(k0, rows0, ssem0)

            @pl.when(i < HC // 2 - 1)
            def _():
                start_gather(k0 + 2, rows0, sem0)

            start_flush(k0 + 1, rows1, ssem1)
            wait_flush(k0 + 1, rows1, ssem1)

            @pl.when(i < HC // 2 - 1)
            def _():
                start_gather(k0 + 3, rows1, sem1)

            return 0

        lax.fori_loop(0, HC // 2, loop_body, 0)
        return 0

    lax.fori_loop(0, 2, half_body, 0)

    plsc.subcore_barrier()
    sl = pl.ds(s * 1000, 1000)

    @pl.when((c == 0) & (s < 10))
    def _():
        pltpu.sync_copy(acc_sh.at[sl], p0_hbm.at[sl])

    @pl.when((c == 1) & (s < 10))
    def _():
        pltpu.sync_copy(acc_sh.at[sl], p1_hbm.at[sl])


def _deg_body(histo_ref, histi_ref, normo_ref, normi_ref):
    dego = jnp.sum(histo_ref[...], axis=0)
    degi = jnp.sum(histi_ref[...], axis=0)
    normo_ref[...] = lax.rsqrt(jnp.maximum(dego, 1.0))
    normi_ref[...] = lax.rsqrt(jnp.maximum(degi, 1.0))


def _degnorm(histo, histi):
    return pl.pallas_call(
        _deg_body,
        out_shape=(jax.ShapeDtypeStruct((HR, F), f32),
                   jax.ShapeDtypeStruct((HR, F), f32)),
    )(histo, histi)


def _mm1_body(x_ref, w_ref, nrm_ref, y_ref):
    y_ref[...] = jnp.dot(x_ref[...], w_ref[0],
                         preferred_element_type=f32) * nrm_ref[0, 0][:, None]


def _mm1(x, w, normo):
    return pl.pallas_call(
        _mm1_body,
        grid=(R, NB),
        in_specs=[
            pl.BlockSpec((BN, F), lambda r, i: (i, 0)),
            pl.BlockSpec((1, F, F), lambda r, i: (r, 0, 0)),
            pl.BlockSpec((1, 1, BN), lambda r, i: (r * NB + i, 0, 0)),
        ],
        out_specs=pl.BlockSpec((BN, F), lambda r, i: (r * NB + i, 0)),
        out_shape=jax.ShapeDtypeStruct((RN, F), f32),
    )(x, w, normo)


def _mm2_body(p0_ref, p1_ref, b1_ref, w_ref, nrm_ref, y_ref):
    h = p0_ref[...] + p1_ref[...] + jnp.sum(b1_ref[...], axis=0)[None, :]
    h = jnp.maximum(h, 0.0)
    y_ref[...] = jnp.dot(h, w_ref[0],
                         preferred_element_type=f32) * nrm_ref[0, 0][:, None]


def _mm2(p0, p1, b1, w2, normo):
    return pl.pallas_call(
        _mm2_body,
        grid=(R, NB),
        in_specs=[
            pl.BlockSpec((BN, F), lambda r, i: (i, 0)),
            pl.BlockSpec((BN, F), lambda r, i: (i, 0)),
            pl.BlockSpec((R, F), lambda r, i: (0, 0)),
            pl.BlockSpec((1, F, F), lambda r, i: (r, 0, 0)),
            pl.BlockSpec((1, 1, BN), lambda r, i: (r * NB + i, 0, 0)),
        ],
        out_specs=pl.BlockSpec((BN, F), lambda r, i: (r * NB + i, 0)),
        out_shape=jax.ShapeDtypeStruct((RN, F), f32),
    )(p0, p1, b1, w2, normo)


def _final_body(p0_ref, p1_ref, b2_ref, o_ref):
    o_ref[...] = (p0_ref[...] + p1_ref[...]
                  + jnp.sum(b2_ref[...], axis=0)[None, :])


def _final(p0, p1, b2):
    return pl.pallas_call(
        _final_body,
        grid=(NB,),
        in_specs=[
            pl.BlockSpec((BN, F), lambda i: (i, 0)),
            pl.BlockSpec((BN, F), lambda i: (i, 0)),
            pl.BlockSpec((R, F), lambda i: (0, 0)),
        ],
        out_specs=pl.BlockSpec((BN, F), lambda i: (i, 0)),
        out_shape=jax.ShapeDtypeStruct((N, F), f32),
    )(p0, p1, b2)


def kernel(features, edge_index, edge_type, W1, b1, W2, b2):
    src = edge_index[0]
    dst = edge_index[1]
    et = edge_type
    zeros1000 = jnp.zeros((1000, F), f32)
    zeros_rn = jnp.zeros((RN,), f32)

    histo, histi, gsrc, gdst = _hist_kernel(src, dst, et, zeros_rn)
    normo625, normi625 = _degnorm(jnp.reshape(histo, (NS, HR, F)),
                                  jnp.reshape(histi, (NS, HR, F)))
    w_e = _edge_weight_kernel(jnp.reshape(normi625, (RN,)), gdst)
    normo = jnp.reshape(normo625, (R * NB, 1, BN))
    gsrc2 = jnp.reshape(gsrc, (E // CH_C, CH_C))
    dst2 = jnp.reshape(dst, (E // CH_C, CH_C))
    w2d = jnp.pad(jnp.reshape(w_e, (E // CH_C, CH_C)),
                  ((0, 0), (0, F - CH_C)))

    y1 = _mm1(features, W1, normo)
    p0, p1 = _edge_pass_kernel(y1, gsrc2, dst2, w2d, zeros1000)
    y2 = _mm2(p0, p1, b1, W2, normo)
    q0, q1 = _edge_pass_kernel(y2, gsrc2, dst2, w2d, zeros1000)
    return _final(q0, q1, b2)
